# baseline (speedup 1.0000x reference)
import functools

import jax
import jax.numpy as jnp
from jax.experimental import pallas as pl
from jax.experimental.pallas import tpu as pltpu

NUM_EXPERTS = 8
TOP_K = 2
DIM = 240
SEQ = 512
BR = 4


def _moe_block(x_ref, gw_ref, gb_ref, ew_ref, eb_ref, o_ref):
    for r in range(BR):
        xbt = x_ref[r]

        logits_t = jax.lax.dot_general(
            gw_ref[...], xbt, (((1,), (0,)), ((), ())),
            preferred_element_type=jnp.float32,
        ) + gb_ref[...]

        sub = jax.lax.broadcasted_iota(jnp.int32, (NUM_EXPERTS, SEQ), 0)
        m1 = jnp.max(logits_t, axis=0, keepdims=True)
        i1 = jnp.min(jnp.where(logits_t == m1, sub, NUM_EXPERTS), axis=0,
                     keepdims=True)
        masked = jnp.where(sub == i1, -jnp.inf, logits_t)
        m2 = jnp.max(masked, axis=0, keepdims=True)
        i2 = jnp.min(jnp.where(masked == m2, sub, NUM_EXPERTS), axis=0,
                     keepdims=True)
        e2 = jnp.exp(m2 - m1)
        w1 = 1.0 / (1.0 + e2)
        w2 = e2 * w1
        wt = jnp.where(sub == i1, w1, 0.0) + jnp.where(sub == i2, w2, 0.0)

        acc = jax.lax.dot_general(
            eb_ref[...], wt, (((0,), (0,)), ((), ())),
            preferred_element_type=jnp.float32,
        )
        for e in range(NUM_EXPERTS):
            yet = jax.lax.dot_general(
                ew_ref[e], xbt, (((1,), (0,)), ((), ())),
                preferred_element_type=jnp.float32,
            )
            acc = acc + wt[e:e + 1, :] * yet
        o_ref[r] = acc


@jax.jit
def kernel(x, gate_w, gate_b, expert_w, expert_b):
    b, s, d = x.shape
    xt = jnp.transpose(x, (0, 2, 1))
    gb2 = gate_b.reshape(NUM_EXPERTS, 1)

    out_t = pl.pallas_call(
        _moe_block,
        grid=(b // BR,),
        in_specs=[
            pl.BlockSpec((BR, d, s), lambda i: (i, 0, 0)),
            pl.BlockSpec((NUM_EXPERTS, d), lambda i: (0, 0)),
            pl.BlockSpec((NUM_EXPERTS, 1), lambda i: (0, 0)),
            pl.BlockSpec((NUM_EXPERTS, d, d), lambda i: (0, 0, 0)),
            pl.BlockSpec((NUM_EXPERTS, d), lambda i: (0, 0)),
        ],
        out_specs=pl.BlockSpec((BR, d, s), lambda i: (i, 0, 0)),
        out_shape=jax.ShapeDtypeStruct((b, d, s), jnp.float32),
        compiler_params=pltpu.CompilerParams(
            dimension_semantics=("arbitrary",),
        ),
    )(xt, gate_w, gb2, expert_w, expert_b)
    return jnp.transpose(out_t, (0, 2, 1))


# speedup vs baseline: 1.0073x; 1.0073x over previous
import functools

import jax
import jax.numpy as jnp
from jax.experimental import pallas as pl
from jax.experimental.pallas import tpu as pltpu

NUM_EXPERTS = 8
TOP_K = 2
DIM = 240
SEQ = 512
BR = 8


def _moe_block(x_ref, gw_ref, gb_ref, ew_ref, eb_ref, o_ref):
    for r in range(BR):
        xbt = x_ref[r]

        logits_t = jax.lax.dot_general(
            gw_ref[...], xbt, (((1,), (0,)), ((), ())),
            preferred_element_type=jnp.float32,
        ) + gb_ref[...]

        sub = jax.lax.broadcasted_iota(jnp.int32, (NUM_EXPERTS, SEQ), 0)
        m1 = jnp.max(logits_t, axis=0, keepdims=True)
        i1 = jnp.min(jnp.where(logits_t == m1, sub, NUM_EXPERTS), axis=0,
                     keepdims=True)
        masked = jnp.where(sub == i1, -jnp.inf, logits_t)
        m2 = jnp.max(masked, axis=0, keepdims=True)
        i2 = jnp.min(jnp.where(masked == m2, sub, NUM_EXPERTS), axis=0,
                     keepdims=True)
        e2 = jnp.exp(m2 - m1)
        w1 = 1.0 / (1.0 + e2)
        w2 = e2 * w1
        wt = jnp.where(sub == i1, w1, 0.0) + jnp.where(sub == i2, w2, 0.0)

        acc = jax.lax.dot_general(
            eb_ref[...], wt, (((0,), (0,)), ((), ())),
            preferred_element_type=jnp.float32,
        )
        for e in range(NUM_EXPERTS):
            yet = jax.lax.dot_general(
                ew_ref[e], xbt, (((1,), (0,)), ((), ())),
                preferred_element_type=jnp.float32,
            )
            acc = acc + wt[e:e + 1, :] * yet
        o_ref[r] = acc


@jax.jit
def kernel(x, gate_w, gate_b, expert_w, expert_b):
    b, s, d = x.shape
    xt = jnp.transpose(x, (0, 2, 1))
    gb2 = gate_b.reshape(NUM_EXPERTS, 1)

    out_t = pl.pallas_call(
        _moe_block,
        grid=(b // BR,),
        in_specs=[
            pl.BlockSpec((BR, d, s), lambda i: (i, 0, 0)),
            pl.BlockSpec((NUM_EXPERTS, d), lambda i: (0, 0)),
            pl.BlockSpec((NUM_EXPERTS, 1), lambda i: (0, 0)),
            pl.BlockSpec((NUM_EXPERTS, d, d), lambda i: (0, 0, 0)),
            pl.BlockSpec((NUM_EXPERTS, d), lambda i: (0, 0)),
        ],
        out_specs=pl.BlockSpec((BR, d, s), lambda i: (i, 0, 0)),
        out_shape=jax.ShapeDtypeStruct((b, d, s), jnp.float32),
        compiler_params=pltpu.CompilerParams(
            dimension_semantics=("parallel",),
        ),
    )(xt, gate_w, gb2, expert_w, expert_b)
    return jnp.transpose(out_t, (0, 2, 1))


# speedup vs baseline: 1.0074x; 1.0001x over previous
import jax
import jax.numpy as jnp
from jax.experimental import pallas as pl
from jax.experimental.pallas import tpu as pltpu

NUM_EXPERTS = 8
TOP_K = 2
DIM = 240
SEQ = 512
BR = 8


def _moe_block(x_ref, gw_ref, gb_ref, ew_ref, eb_ref, o_ref):
    for r in range(BR):
        xbt = x_ref[r]

        logits_t = jax.lax.dot_general(
            gw_ref[...], xbt, (((1,), (0,)), ((), ())),
            preferred_element_type=jnp.float32,
        ) + gb_ref[:, 0:1]

        sub = jax.lax.broadcasted_iota(jnp.int32, (NUM_EXPERTS, SEQ), 0)
        m1 = jnp.max(logits_t, axis=0, keepdims=True)
        i1 = jnp.min(jnp.where(logits_t == m1, sub, NUM_EXPERTS), axis=0,
                     keepdims=True)
        masked = jnp.where(sub == i1, -jnp.inf, logits_t)
        m2 = jnp.max(masked, axis=0, keepdims=True)
        i2 = jnp.min(jnp.where(masked == m2, sub, NUM_EXPERTS), axis=0,
                     keepdims=True)
        e2 = jnp.exp(m2 - m1)
        w1 = 1.0 / (1.0 + e2)
        w2 = e2 * w1
        wt = jnp.where(sub == i1, w1, 0.0) + jnp.where(sub == i2, w2, 0.0)

        acc = jax.lax.dot_general(
            eb_ref[...], wt, (((0,), (0,)), ((), ())),
            preferred_element_type=jnp.float32,
        )
        for e in range(NUM_EXPERTS):
            yet = jax.lax.dot_general(
                ew_ref[e], xbt, (((1,), (0,)), ((), ())),
                preferred_element_type=jnp.float32,
            )
            acc = acc + wt[e:e + 1, :] * yet
        o_ref[r] = acc


@jax.jit
def kernel(x, gate_w, gate_b, expert_w, expert_b):
    b, s, d = x.shape
    xt = jnp.transpose(x, (0, 2, 1))
    gb2 = jnp.broadcast_to(gate_b.reshape(NUM_EXPERTS, 1), (NUM_EXPERTS, 128))

    out_t = pl.pallas_call(
        _moe_block,
        grid=(b // BR,),
        in_specs=[
            pl.BlockSpec((BR, d, s), lambda i: (i, 0, 0)),
            pl.BlockSpec((NUM_EXPERTS, d), lambda i: (0, 0)),
            pl.BlockSpec((NUM_EXPERTS, 128), lambda i: (0, 0)),
            pl.BlockSpec((NUM_EXPERTS, d, d), lambda i: (0, 0, 0)),
            pl.BlockSpec((NUM_EXPERTS, d), lambda i: (0, 0)),
        ],
        out_specs=pl.BlockSpec((BR, d, s), lambda i: (i, 0, 0)),
        out_shape=jax.ShapeDtypeStruct((b, d, s), jnp.float32),
        compiler_params=pltpu.CompilerParams(
            dimension_semantics=("parallel",),
        ),
    )(xt, gate_w, gb2, expert_w, expert_b)
    return jnp.transpose(out_t, (0, 2, 1))
